# baseline (speedup 1.0000x reference)
import functools

import numpy as np
import jax
import jax.numpy as jnp
from jax.experimental import pallas as pl
from jax.experimental.pallas import tpu as pltpu

_LANE = 128


def _round_up_lanes(n):
    return (n + _LANE - 1) // _LANE * _LANE


def _tap_shifts_and_masks(H, W, ksize, padding):
    q = np.arange(H * W)
    h, w = q // W, q % W
    shifts, masks = [], []
    for i in range(ksize):
        for j in range(ksize):
            hh, ww = h + i - padding, w + j - padding
            shifts.append((i - padding) * W + (j - padding))
            masks.append((hh >= 0) & (hh < H) & (ww >= 0) & (ww < W))
    return shifts, np.stack(masks).astype(np.float32)


def _conv_stats_kernel(x_ref, w_ref, b_ref, m_ref, y_ref, stat_ref, *, shifts):
    hw = x_ref.shape[-1]
    for n in range(x_ref.shape[0]):
        xb = x_ref[n].astype(jnp.bfloat16)
        pieces = []
        for t, d in enumerate(shifts):
            if d == 0:
                xs = xb
            else:
                s = d % hw
                xs = jnp.concatenate([xb[:, s:], xb[:, :s]], axis=1)
            pieces.append(xs * m_ref[t:t + 1, :])
        patch = jnp.concatenate(pieces, axis=0)
        y = jnp.dot(w_ref[...], patch,
                    preferred_element_type=jnp.float32)
        y = y + b_ref[...]
        st = jnp.concatenate(
            [jnp.sum(y, axis=1, keepdims=True),
             jnp.sum(y * y, axis=1, keepdims=True)], axis=0)
        stat_ref[n] = jnp.broadcast_to(st, (st.shape[0], _LANE))
        y_ref[n, :, :hw] = y.astype(jnp.bfloat16)


def _bn_hswish_kernel(y_ref, scale_ref, shift_ref, out_ref):
    hw = out_ref.shape[-1]
    yb = y_ref[:, :, :hw].astype(jnp.float32) * scale_ref[...] + shift_ref[...]
    out_ref[...] = yb * jnp.clip(yb + 3.0, 0.0, 6.0) * (1.0 / 6.0)


@functools.partial(jax.jit, static_argnames=("ksize", "padding"))
def _conv_block(x, weight, bias, gamma, beta, *, ksize=3, padding=1):
    N, Cin, H, W = x.shape
    Cout = weight.shape[0]
    HW = H * W
    HWp = _round_up_lanes(HW)
    ntaps = ksize * ksize

    x_flat = x.reshape(N, Cin, HW).astype(jnp.float32)

    w_all = jnp.transpose(weight.astype(jnp.float32), (0, 2, 3, 1))
    w_all = w_all.reshape(Cout, ntaps * Cin).astype(jnp.bfloat16)
    b_col = bias.astype(jnp.float32).reshape(Cout, 1)

    shifts, masks_np = _tap_shifts_and_masks(H, W, ksize, padding)
    masks = jnp.asarray(masks_np, dtype=jnp.bfloat16)

    na = 2 if N % 2 == 0 else 1
    kern1 = functools.partial(_conv_stats_kernel, shifts=shifts)
    y_pad, pstat = pl.pallas_call(
        kern1,
        out_shape=(
            jax.ShapeDtypeStruct((N, Cout, HWp), jnp.bfloat16),
            jax.ShapeDtypeStruct((N, 2 * Cout, _LANE), jnp.float32),
        ),
        grid=(N // na,),
        in_specs=[
            pl.BlockSpec((na, Cin, HW), lambda n: (n, 0, 0)),
            pl.BlockSpec((Cout, ntaps * Cin), lambda n: (0, 0)),
            pl.BlockSpec((Cout, 1), lambda n: (0, 0)),
            pl.BlockSpec((ntaps, HW), lambda n: (0, 0)),
        ],
        out_specs=(
            pl.BlockSpec((na, Cout, HWp), lambda n: (n, 0, 0)),
            pl.BlockSpec((na, 2 * Cout, _LANE), lambda n: (n, 0, 0)),
        ),
        compiler_params=pltpu.CompilerParams(
            dimension_semantics=("parallel",)),
    )(x_flat, w_all, b_col, masks)

    cnt = jnp.float32(N * HW)
    s = jnp.sum(pstat[:, :Cout, 0], axis=0)
    ss = jnp.sum(pstat[:, Cout:, 0], axis=0)
    mean = s / cnt
    var = jnp.maximum(ss / cnt - mean * mean, 0.0)
    inv = jax.lax.rsqrt(var + 1e-5)
    g = gamma.astype(jnp.float32)
    scale = (g * inv).reshape(Cout, 1)
    shift = (beta.astype(jnp.float32) - mean * g * inv).reshape(Cout, 1)

    nb = 4 if N % 4 == 0 else 1
    out_flat = pl.pallas_call(
        _bn_hswish_kernel,
        out_shape=jax.ShapeDtypeStruct((N, Cout, HW), jnp.float32),
        grid=(N // nb,),
        in_specs=[
            pl.BlockSpec((nb, Cout, HWp), lambda n: (n, 0, 0)),
            pl.BlockSpec((Cout, 1), lambda n: (0, 0)),
            pl.BlockSpec((Cout, 1), lambda n: (0, 0)),
        ],
        out_specs=pl.BlockSpec((nb, Cout, HW), lambda n: (n, 0, 0)),
        compiler_params=pltpu.CompilerParams(
            dimension_semantics=("parallel",)),
    )(y_pad, scale, shift)

    return out_flat.reshape(N, Cout, H, W)


def kernel(x, weight, bias, gamma, beta):
    return _conv_block(x, weight, bias, gamma, beta, ksize=3, padding=1)


# speedup vs baseline: 1.0532x; 1.0532x over previous
import functools

import numpy as np
import jax
import jax.numpy as jnp
from jax.experimental import pallas as pl
from jax.experimental.pallas import tpu as pltpu

_LANE = 128


def _tap_shifts_and_masks(H, W, ksize, padding):
    q = np.arange(H * W)
    h, w = q // W, q % W
    shifts, masks = [], []
    for i in range(ksize):
        for j in range(ksize):
            hh, ww = h + i - padding, w + j - padding
            shifts.append((i - padding) * W + (j - padding))
            masks.append((hh >= 0) & (hh < H) & (ww >= 0) & (ww < W))
    return shifts, np.stack(masks).astype(np.float32)


def _fused_kernel(x_ref, w_ref, b_ref, m_ref, g_ref, be_ref, out_ref,
                  y_scr, st_scr, *, shifts, n_imgs, cnt):
    n = pl.program_id(0)
    hw = x_ref.shape[-1]
    c = b_ref.shape[0]

    @pl.when(n < n_imgs)
    def _conv_phase():
        xb = x_ref[0].astype(jnp.bfloat16)
        pieces = []
        for t, d in enumerate(shifts):
            if d == 0:
                xs = xb
            else:
                s = d % hw
                xs = jnp.concatenate([xb[:, s:], xb[:, :s]], axis=1)
            pieces.append(xs * m_ref[t:t + 1, :])
        patch = jnp.concatenate(pieces, axis=0)
        y = jnp.dot(w_ref[...], patch,
                    preferred_element_type=jnp.float32)
        y = y + b_ref[...]
        st = jnp.concatenate(
            [jnp.sum(y, axis=1, keepdims=True),
             jnp.sum(y * y, axis=1, keepdims=True)], axis=0)
        stb = jnp.broadcast_to(st, (2 * c, _LANE))
        st_scr[...] = jnp.where(n == 0, stb, st_scr[...] + stb)
        y_scr[pl.ds(n, 1)] = y.astype(jnp.bfloat16)[None]

    @pl.when(n >= n_imgs)
    def _bn_hswish_phase():
        p = st_scr[:, 0:1]
        mean = p[:c] * (1.0 / cnt)
        var = jnp.maximum(p[c:] * (1.0 / cnt) - mean * mean, 0.0)
        inv = jax.lax.rsqrt(var + 1e-5)
        scale = g_ref[...] * inv
        shift = be_ref[...] - mean * scale
        yv = y_scr[pl.ds(n - n_imgs, 1)][0]
        yb = yv.astype(jnp.float32) * scale + shift
        out_ref[0] = yb * jnp.clip(yb + 3.0, 0.0, 6.0) * (1.0 / 6.0)


@functools.partial(jax.jit, static_argnames=("ksize", "padding"))
def _conv_block(x, weight, bias, gamma, beta, *, ksize=3, padding=1):
    N, Cin, H, W = x.shape
    Cout = weight.shape[0]
    HW = H * W
    ntaps = ksize * ksize

    x_flat = x.reshape(N, Cin, HW).astype(jnp.float32)

    w_all = jnp.transpose(weight.astype(jnp.float32), (0, 2, 3, 1))
    w_all = w_all.reshape(Cout, ntaps * Cin).astype(jnp.bfloat16)
    b_col = bias.astype(jnp.float32).reshape(Cout, 1)
    g_col = gamma.astype(jnp.float32).reshape(Cout, 1)
    be_col = beta.astype(jnp.float32).reshape(Cout, 1)

    shifts, masks_np = _tap_shifts_and_masks(H, W, ksize, padding)
    masks = jnp.asarray(masks_np, dtype=jnp.bfloat16)

    kern = functools.partial(_fused_kernel, shifts=shifts, n_imgs=N,
                             cnt=float(N * HW))
    out_flat = pl.pallas_call(
        kern,
        out_shape=jax.ShapeDtypeStruct((N, Cout, HW), jnp.float32),
        grid=(2 * N,),
        in_specs=[
            pl.BlockSpec((1, Cin, HW),
                         lambda n: (jnp.minimum(n, N - 1), 0, 0)),
            pl.BlockSpec((Cout, ntaps * Cin), lambda n: (0, 0)),
            pl.BlockSpec((Cout, 1), lambda n: (0, 0)),
            pl.BlockSpec((ntaps, HW), lambda n: (0, 0)),
            pl.BlockSpec((Cout, 1), lambda n: (0, 0)),
            pl.BlockSpec((Cout, 1), lambda n: (0, 0)),
        ],
        out_specs=pl.BlockSpec((1, Cout, HW),
                               lambda n: (jnp.maximum(n - N, 0), 0, 0)),
        scratch_shapes=[
            pltpu.VMEM((N, Cout, HW), jnp.bfloat16),
            pltpu.VMEM((2 * Cout, _LANE), jnp.float32),
        ],
        compiler_params=pltpu.CompilerParams(
            dimension_semantics=("arbitrary",)),
    )(x_flat, w_all, b_col, masks, g_col, be_col)

    return out_flat.reshape(N, Cout, H, W)


def kernel(x, weight, bias, gamma, beta):
    return _conv_block(x, weight, bias, gamma, beta, ksize=3, padding=1)


# speedup vs baseline: 1.0539x; 1.0007x over previous
import functools

import numpy as np
import jax
import jax.numpy as jnp
from jax.experimental import pallas as pl
from jax.experimental.pallas import tpu as pltpu

_LANE = 128


def _tap_shifts_and_masks(H, W, ksize, padding):
    q = np.arange(H * W)
    h, w = q // W, q % W
    shifts, masks = [], []
    for i in range(ksize):
        for j in range(ksize):
            hh, ww = h + i - padding, w + j - padding
            shifts.append((i - padding) * W + (j - padding))
            masks.append((hh >= 0) & (hh < H) & (ww >= 0) & (ww < W))
    return shifts, np.stack(masks).astype(np.float32)


def _fused_kernel(x_ref, w_ref, b_ref, m_ref, g_ref, be_ref, out_ref,
                  y_scr, st_scr, *, shifts, n_imgs, cnt):
    n = pl.program_id(0)
    hw = x_ref.shape[-1]
    c = b_ref.shape[0]
    nb = out_ref.shape[0]

    @pl.when(n < n_imgs)
    def _conv_phase():
        xb = x_ref[0].astype(jnp.bfloat16)
        pieces = []
        for t, d in enumerate(shifts):
            if d == 0:
                xs = xb
            else:
                s = d % hw
                xs = jnp.concatenate([xb[:, s:], xb[:, :s]], axis=1)
            pieces.append(xs * m_ref[t:t + 1, :])
        patch = jnp.concatenate(pieces, axis=0)
        y = jnp.dot(w_ref[...], patch,
                    preferred_element_type=jnp.float32)
        y = y + b_ref[...]
        st = jnp.concatenate(
            [jnp.sum(y, axis=1, keepdims=True),
             jnp.sum(y * y, axis=1, keepdims=True)], axis=0)
        stb = jnp.broadcast_to(st, (2 * c, _LANE))
        st_scr[...] = jnp.where(n == 0, stb, st_scr[...] + stb)
        y_scr[pl.ds(n, 1)] = y.astype(jnp.bfloat16)[None]

    @pl.when(n >= n_imgs)
    def _bn_hswish_phase():
        p = st_scr[:, 0:1]
        mean = p[:c] * (1.0 / cnt)
        var = jnp.maximum(p[c:] * (1.0 / cnt) - mean * mean, 0.0)
        inv = jax.lax.rsqrt(var + 1e-5)
        scale = g_ref[...] * inv
        shift = be_ref[...] - mean * scale
        yv = y_scr[pl.ds((n - n_imgs) * nb, nb)]
        yb = yv.astype(jnp.float32) * scale + shift
        out_ref[...] = yb * jnp.clip(yb + 3.0, 0.0, 6.0) * (1.0 / 6.0)


@functools.partial(jax.jit, static_argnames=("ksize", "padding"))
def _conv_block(x, weight, bias, gamma, beta, *, ksize=3, padding=1):
    N, Cin, H, W = x.shape
    Cout = weight.shape[0]
    HW = H * W
    ntaps = ksize * ksize

    x_flat = x.reshape(N, Cin, HW).astype(jnp.float32)

    w_all = jnp.transpose(weight.astype(jnp.float32), (0, 2, 3, 1))
    w_all = w_all.reshape(Cout, ntaps * Cin).astype(jnp.bfloat16)
    b_col = bias.astype(jnp.float32).reshape(Cout, 1)
    g_col = gamma.astype(jnp.float32).reshape(Cout, 1)
    be_col = beta.astype(jnp.float32).reshape(Cout, 1)

    shifts, masks_np = _tap_shifts_and_masks(H, W, ksize, padding)
    masks = jnp.asarray(masks_np, dtype=jnp.bfloat16)

    nb = 4 if N % 4 == 0 else 1
    kern = functools.partial(_fused_kernel, shifts=shifts, n_imgs=N,
                             cnt=float(N * HW))
    out_flat = pl.pallas_call(
        kern,
        out_shape=jax.ShapeDtypeStruct((N, Cout, HW), jnp.float32),
        grid=(N + N // nb,),
        in_specs=[
            pl.BlockSpec((1, Cin, HW),
                         lambda n: (jnp.minimum(n, N - 1), 0, 0)),
            pl.BlockSpec((Cout, ntaps * Cin), lambda n: (0, 0)),
            pl.BlockSpec((Cout, 1), lambda n: (0, 0)),
            pl.BlockSpec((ntaps, HW), lambda n: (0, 0)),
            pl.BlockSpec((Cout, 1), lambda n: (0, 0)),
            pl.BlockSpec((Cout, 1), lambda n: (0, 0)),
        ],
        out_specs=pl.BlockSpec((nb, Cout, HW),
                               lambda n: (jnp.maximum(n - N, 0), 0, 0)),
        scratch_shapes=[
            pltpu.VMEM((N, Cout, HW), jnp.bfloat16),
            pltpu.VMEM((2 * Cout, _LANE), jnp.float32),
        ],
        compiler_params=pltpu.CompilerParams(
            dimension_semantics=("arbitrary",)),
    )(x_flat, w_all, b_col, masks, g_col, be_col)

    return out_flat.reshape(N, Cout, H, W)


def kernel(x, weight, bias, gamma, beta):
    return _conv_block(x, weight, bias, gamma, beta, ksize=3, padding=1)
